# initial kernel scaffold (unmeasured)
import jax
import jax.numpy as jnp
from jax import lax
from jax.experimental import pallas as pl
from jax.experimental.pallas import tpu as pltpu

N_DEV = 8
SQ_PER = 256
QBLK = 64
HQ_PER = 8
DH = 128
SKV = 4096
DM = 1024
SCALE = 0.08838834764831843
F32 = jnp.float32


def kernel(x, Wq, K_ext, V_ext, Wo):
    def body(x_ref, wq_ref, k_hbm, v_hbm, wo_ref, out_ref,
             xg, kq, vq, acc, sendbuf, recvbuf,
             ag_send, ag_recv, rs_send, rs_recv, kv_sems):
        my = lax.axis_index("i")
        left = lax.rem(my - 1 + N_DEV, N_DEV)
        right = lax.rem(my + 1, N_DEV)

        h0 = my * HQ_PER
        kv_copies = []
        for h in range(HQ_PER):
            ck = pltpu.make_async_copy(
                k_hbm.at[0, :, h0 + h, :], kq.at[h], kv_sems.at[h])
            cv = pltpu.make_async_copy(
                v_hbm.at[0, :, h0 + h, :], vq.at[h], kv_sems.at[HQ_PER + h])
            ck.start()
            cv.start()
            kv_copies.append(ck)
            kv_copies.append(cv)

        xg[pl.ds(my, 1)] = x_ref[:]

        barrier = pltpu.get_barrier_semaphore()
        for nbr in (left, right):
            pl.semaphore_signal(barrier, inc=1, device_id=(nbr,),
                                device_id_type=pl.DeviceIdType.MESH)
        pl.semaphore_wait(barrier, 2)

        for h in range(N_DEV - 1):
            o = lax.rem(my - h + N_DEV, N_DEV)
            rdma = pltpu.make_async_remote_copy(
                src_ref=xg.at[pl.ds(o, 1)],
                dst_ref=xg.at[pl.ds(o, 1)],
                send_sem=ag_send.at[h],
                recv_sem=ag_recv.at[h],
                device_id=(right,),
                device_id_type=pl.DeviceIdType.MESH,
            )
            rdma.start()
            rdma.wait()

        for c in kv_copies:
            c.wait()

        wq = wq_ref[:]
        wo = wo_ref[:]
        kb = lax.broadcasted_iota(jnp.int32, (SQ_PER, SKV), 1) // QBLK
        for o in range(N_DEV):
            xt = xg[o]
            q = jnp.dot(xt, wq, preferred_element_type=F32)
            qb = (o * (SQ_PER // QBLK)
                  + lax.broadcasted_iota(jnp.int32, (SQ_PER, SKV), 0) // QBLK)
            mask = (qb == kb) | (kb == 0) | (lax.rem(qb + kb, 3) == 0)
            ctx_cols = []
            for h in range(HQ_PER):
                qh = q[:, h * DH:(h + 1) * DH]
                s = lax.dot_general(
                    qh, kq[h], (((1,), (1,)), ((), ())),
                    preferred_element_type=F32) * SCALE
                w = jnp.where(mask, jnp.exp(s), 0.0)
                wsum = jnp.sum(w, axis=1, keepdims=True)
                ctx_h = jnp.dot(w, vq[h], preferred_element_type=F32) / wsum
                ctx_cols.append(ctx_h)
            ctx = jnp.concatenate(ctx_cols, axis=1)
            acc[o] = jnp.dot(ctx, wo, preferred_element_type=F32)

        for t in range(N_DEV - 1):
            o = lax.rem(my - t - 1 + N_DEV, N_DEV)
            data = acc[pl.ds(o, 1)]
            if t > 0:
                data = data + recvbuf[pl.ds(t - 1, 1)]
            sendbuf[:] = data[0]
            rdma = pltpu.make_async_remote_copy(
                src_ref=sendbuf,
                dst_ref=recvbuf.at[t],
                send_sem=rs_send.at[t],
                recv_sem=rs_recv.at[t],
                device_id=(right,),
                device_id_type=pl.DeviceIdType.MESH,
            )
            rdma.start()
            rdma.wait()

        out_ref[:] = acc[pl.ds(my, 1)] + recvbuf[pl.ds(N_DEV - 2, 1)]

    return pl.pallas_call(
        body,
        out_shape=jax.ShapeDtypeStruct((1, SQ_PER, DM), F32),
        in_specs=[
            pl.BlockSpec(memory_space=pltpu.VMEM),
            pl.BlockSpec(memory_space=pltpu.VMEM),
            pl.BlockSpec(memory_space=pltpu.ANY),
            pl.BlockSpec(memory_space=pltpu.ANY),
            pl.BlockSpec(memory_space=pltpu.VMEM),
        ],
        out_specs=pl.BlockSpec(memory_space=pltpu.VMEM),
        scratch_shapes=[
            pltpu.VMEM((N_DEV, SQ_PER, DM), F32),
            pltpu.VMEM((HQ_PER, SKV, DH), F32),
            pltpu.VMEM((HQ_PER, SKV, DH), F32),
            pltpu.VMEM((N_DEV, SQ_PER, DM), F32),
            pltpu.VMEM((SQ_PER, DM), F32),
            pltpu.VMEM((N_DEV - 1, SQ_PER, DM), F32),
            pltpu.SemaphoreType.DMA((N_DEV - 1,)),
            pltpu.SemaphoreType.DMA((N_DEV - 1,)),
            pltpu.SemaphoreType.DMA((N_DEV - 1,)),
            pltpu.SemaphoreType.DMA((N_DEV - 1,)),
            pltpu.SemaphoreType.DMA((2 * HQ_PER,)),
        ],
        compiler_params=pltpu.CompilerParams(collective_id=0),
    )(x, Wq, K_ext, V_ext, Wo)


# baseline (device time: 434136 ns/iter reference)
import jax
import jax.numpy as jnp
from jax import lax
from jax.experimental import pallas as pl
from jax.experimental.pallas import tpu as pltpu

N_DEV = 8
SQ_PER = 256
QBLK = 64
HQ_PER = 8
DH = 128
SKV = 4096
DM = 1024
SCALE = 0.08838834764831843
F32 = jnp.float32


def kernel(x, Wq, K_ext, V_ext, Wo):
    def body(x_ref, wq_ref, k_hbm, v_hbm, wo_ref, out_ref,
             xg, qall, acc, kbuf, vbuf, sendbuf, recvbuf,
             ag_send, ag_recv, rs_send, rs_recv, kv_sems):
        my = lax.axis_index("i")
        left = lax.rem(my - 1 + N_DEV, N_DEV)
        right = lax.rem(my + 1, N_DEV)
        h0 = my * HQ_PER

        def kv_dma(h, slot):
            ck = pltpu.make_async_copy(
                k_hbm.at[0, :, h0 + h, :], kbuf.at[slot], kv_sems.at[slot, 0])
            cv = pltpu.make_async_copy(
                v_hbm.at[0, :, h0 + h, :], vbuf.at[slot], kv_sems.at[slot, 1])
            return ck, cv

        k0, v0 = kv_dma(0, 0)
        k0.start()
        v0.start()

        xg[pl.ds(my, 1)] = x_ref[:]

        barrier = pltpu.get_barrier_semaphore()
        for nbr in (left, right):
            pl.semaphore_signal(barrier, inc=1, device_id=(nbr,),
                                device_id_type=pl.DeviceIdType.MESH)
        pl.semaphore_wait(barrier, 2)

        for h in range(N_DEV - 1):
            o = lax.rem(my - h + N_DEV, N_DEV)
            rdma = pltpu.make_async_remote_copy(
                src_ref=xg.at[pl.ds(o, 1)],
                dst_ref=xg.at[pl.ds(o, 1)],
                send_sem=ag_send.at[h],
                recv_sem=ag_recv.at[h],
                device_id=(right,),
                device_id_type=pl.DeviceIdType.MESH,
            )
            rdma.start()
            rdma.wait()

        def qstep(o, carry):
            qall[o] = jnp.dot(xg[o], wq_ref[:], preferred_element_type=F32)
            return carry

        lax.fori_loop(0, N_DEV, qstep, 0)

        pending = (k0, v0)
        for h in range(HQ_PER):
            slot = h % 2
            pending[0].wait()
            pending[1].wait()
            if h + 1 < HQ_PER:
                kn, vn = kv_dma(h + 1, 1 - slot)
                kn.start()
                vn.start()
                pending = (kn, vn)

            def ostep(o, carry, h=h, slot=slot):
                qh = qall[o, :, h * DH:(h + 1) * DH]
                qb = (o * (SQ_PER // QBLK)
                      + lax.broadcasted_iota(jnp.int32, (SQ_PER, SKV), 0)
                      // QBLK)
                kb = lax.broadcasted_iota(jnp.int32, (SQ_PER, SKV), 1) // QBLK
                mask = (qb == kb) | (kb == 0) | (lax.rem(qb + kb, 3) == 0)
                s = lax.dot_general(
                    qh, kbuf[slot], (((1,), (1,)), ((), ())),
                    preferred_element_type=F32) * SCALE
                w = jnp.where(mask, jnp.exp(s), 0.0)
                wsum = jnp.sum(w, axis=1, keepdims=True)
                ctx_h = jnp.dot(w, vbuf[slot], preferred_element_type=F32) / wsum
                part = jnp.dot(ctx_h, wo_ref[h * DH:(h + 1) * DH, :],
                               preferred_element_type=F32)
                if h == 0:
                    acc[o] = part
                else:
                    acc[o] = acc[o] + part
                return carry

            lax.fori_loop(0, N_DEV, ostep, 0)

        for t in range(N_DEV - 1):
            o = lax.rem(my - t - 1 + N_DEV, N_DEV)
            data = acc[pl.ds(o, 1)]
            if t > 0:
                data = data + recvbuf[pl.ds(t - 1, 1)]
            sendbuf[:] = data[0]
            rdma = pltpu.make_async_remote_copy(
                src_ref=sendbuf,
                dst_ref=recvbuf.at[t],
                send_sem=rs_send.at[t],
                recv_sem=rs_recv.at[t],
                device_id=(right,),
                device_id_type=pl.DeviceIdType.MESH,
            )
            rdma.start()
            rdma.wait()

        out_ref[:] = acc[pl.ds(my, 1)] + recvbuf[pl.ds(N_DEV - 2, 1)]

    return pl.pallas_call(
        body,
        out_shape=jax.ShapeDtypeStruct((1, SQ_PER, DM), F32),
        in_specs=[
            pl.BlockSpec(memory_space=pltpu.VMEM),
            pl.BlockSpec(memory_space=pltpu.VMEM),
            pl.BlockSpec(memory_space=pl.ANY),
            pl.BlockSpec(memory_space=pl.ANY),
            pl.BlockSpec(memory_space=pltpu.VMEM),
        ],
        out_specs=pl.BlockSpec(memory_space=pltpu.VMEM),
        scratch_shapes=[
            pltpu.VMEM((N_DEV, SQ_PER, DM), F32),
            pltpu.VMEM((N_DEV, SQ_PER, DM), F32),
            pltpu.VMEM((N_DEV, SQ_PER, DM), F32),
            pltpu.VMEM((2, SKV, DH), F32),
            pltpu.VMEM((2, SKV, DH), F32),
            pltpu.VMEM((SQ_PER, DM), F32),
            pltpu.VMEM((N_DEV - 1, SQ_PER, DM), F32),
            pltpu.SemaphoreType.DMA((N_DEV - 1,)),
            pltpu.SemaphoreType.DMA((N_DEV - 1,)),
            pltpu.SemaphoreType.DMA((N_DEV - 1,)),
            pltpu.SemaphoreType.DMA((N_DEV - 1,)),
            pltpu.SemaphoreType.DMA((2, 2)),
        ],
        compiler_params=pltpu.CompilerParams(
            collective_id=0,
            vmem_limit_bytes=60 * 1024 * 1024,
        ),
    )(x, Wq, K_ext, V_ext, Wo)


# device time: 321967 ns/iter; 1.3484x vs baseline; 1.3484x over previous
import jax
import jax.numpy as jnp
from jax import lax
from jax.experimental import pallas as pl
from jax.experimental.pallas import tpu as pltpu

N_DEV = 8
SQ_PER = 256
QBLK = 64
HQ_PER = 8
DH = 128
SKV = 4096
DM = 1024
SCALE = 0.08838834764831843
F32 = jnp.float32
BF16 = jnp.bfloat16


def kernel(x, Wq, K_ext, V_ext, Wo):
    def body(x_ref, wq_ref, k_hbm, v_hbm, wo_ref, out_ref,
             xg, acc, kbuf, vbuf, qbuf, sendbuf, recvbuf,
             ag_send, ag_recv, rs_send, rs_recv, kv_sems):
        my = lax.axis_index("i")
        left = lax.rem(my - 1 + N_DEV, N_DEV)
        right = lax.rem(my + 1, N_DEV)
        h0 = my * HQ_PER

        def kv_dma(h, slot):
            ck = pltpu.make_async_copy(
                k_hbm.at[0, :, h0 + h, :], kbuf.at[slot], kv_sems.at[slot, 0])
            cv = pltpu.make_async_copy(
                v_hbm.at[0, :, h0 + h, :], vbuf.at[slot], kv_sems.at[slot, 1])
            return ck, cv

        k0, v0 = kv_dma(0, 0)
        k0.start()
        v0.start()

        xg[pl.ds(my, 1)] = x_ref[:].astype(BF16)

        barrier = pltpu.get_barrier_semaphore()
        for nbr in (left, right):
            pl.semaphore_signal(barrier, inc=1, device_id=(nbr,),
                                device_id_type=pl.DeviceIdType.MESH)
        pl.semaphore_wait(barrier, 2)

        for h in range(N_DEV - 1):
            o = lax.rem(my - h + N_DEV, N_DEV)
            rdma = pltpu.make_async_remote_copy(
                src_ref=xg.at[pl.ds(o, 1)],
                dst_ref=xg.at[pl.ds(o, 1)],
                send_sem=ag_send.at[h],
                recv_sem=ag_recv.at[h],
                device_id=(right,),
                device_id_type=pl.DeviceIdType.MESH,
            )
            rdma.start()
            rdma.wait()

        wqb = wq_ref[:].astype(BF16)
        kb = lax.broadcasted_iota(jnp.int32, (SQ_PER, SKV), 1) // QBLK

        for o in range(N_DEV):
            qbuf[:] = jnp.dot(xg[o], wqb, preferred_element_type=F32
                              ).astype(BF16)
            qb = (o * (SQ_PER // QBLK)
                  + lax.broadcasted_iota(jnp.int32, (SQ_PER, SKV), 0) // QBLK)
            maskf = jnp.where(
                (qb == kb) | (kb == 0) | (lax.rem(qb + kb, 3) == 0),
                1.0, 0.0).astype(F32)
            acc[o] = jnp.zeros((SQ_PER, DM), F32)

            def hstep(h, carry, o=o, maskf=maskf):
                slot = lax.rem(h, 2)
                ck, cv = kv_dma(h, slot)
                ck.wait()
                cv.wait()
                if o < N_DEV - 1:
                    hn = lax.rem(h + 1, HQ_PER)
                    ckn, cvn = kv_dma(hn, 1 - slot)
                    ckn.start()
                    cvn.start()
                else:
                    @pl.when(h < HQ_PER - 1)
                    def _():
                        ckn, cvn = kv_dma(h + 1, 1 - slot)
                        ckn.start()
                        cvn.start()
                kh = kbuf[slot].astype(BF16)
                vh = vbuf[slot].astype(BF16)
                qh = qbuf[:, pl.ds(h * DH, DH)]
                s = lax.dot_general(
                    qh, kh, (((1,), (1,)), ((), ())),
                    preferred_element_type=F32) * SCALE
                w = jnp.exp(s) * maskf
                wsum = jnp.sum(w, axis=1, keepdims=True)
                ctx_h = jnp.dot(w.astype(BF16), vh,
                                preferred_element_type=F32) / wsum
                woh = wo_ref[pl.ds(h * DH, DH), :].astype(BF16)
                part = jnp.dot(ctx_h.astype(BF16), woh,
                               preferred_element_type=F32)
                acc[o] = acc[o] + part
                return carry

            lax.fori_loop(0, HQ_PER, hstep, 0)

        for t in range(N_DEV - 1):
            o = lax.rem(my - t - 1 + N_DEV, N_DEV)
            data = acc[pl.ds(o, 1)]
            if t > 0:
                data = data + recvbuf[pl.ds(t - 1, 1)]
            sendbuf[:] = data[0]
            rdma = pltpu.make_async_remote_copy(
                src_ref=sendbuf,
                dst_ref=recvbuf.at[t],
                send_sem=rs_send.at[t],
                recv_sem=rs_recv.at[t],
                device_id=(right,),
                device_id_type=pl.DeviceIdType.MESH,
            )
            rdma.start()
            rdma.wait()

        out_ref[:] = acc[pl.ds(my, 1)] + recvbuf[pl.ds(N_DEV - 2, 1)]

    return pl.pallas_call(
        body,
        out_shape=jax.ShapeDtypeStruct((1, SQ_PER, DM), F32),
        in_specs=[
            pl.BlockSpec(memory_space=pltpu.VMEM),
            pl.BlockSpec(memory_space=pltpu.VMEM),
            pl.BlockSpec(memory_space=pl.ANY),
            pl.BlockSpec(memory_space=pl.ANY),
            pl.BlockSpec(memory_space=pltpu.VMEM),
        ],
        out_specs=pl.BlockSpec(memory_space=pltpu.VMEM),
        scratch_shapes=[
            pltpu.VMEM((N_DEV, SQ_PER, DM), BF16),
            pltpu.VMEM((N_DEV, SQ_PER, DM), F32),
            pltpu.VMEM((2, SKV, DH), F32),
            pltpu.VMEM((2, SKV, DH), F32),
            pltpu.VMEM((SQ_PER, DM), BF16),
            pltpu.VMEM((SQ_PER, DM), F32),
            pltpu.VMEM((N_DEV - 1, SQ_PER, DM), F32),
            pltpu.SemaphoreType.DMA((N_DEV - 1,)),
            pltpu.SemaphoreType.DMA((N_DEV - 1,)),
            pltpu.SemaphoreType.DMA((N_DEV - 1,)),
            pltpu.SemaphoreType.DMA((N_DEV - 1,)),
            pltpu.SemaphoreType.DMA((2, 2)),
        ],
        compiler_params=pltpu.CompilerParams(
            collective_id=0,
            vmem_limit_bytes=60 * 1024 * 1024,
        ),
    )(x, Wq, K_ext, V_ext, Wo)


# device time: 194492 ns/iter; 2.2322x vs baseline; 1.6554x over previous
import jax
import jax.numpy as jnp
from jax import lax
from jax.experimental import pallas as pl
from jax.experimental.pallas import tpu as pltpu

N_DEV = 8
SQ_PER = 256
QBLK = 64
HQ_PER = 8
DH = 128
SKV = 4096
DM = 1024
SCALE = 0.08838834764831843
F32 = jnp.float32
BF16 = jnp.bfloat16


def kernel(x, Wq, K_ext, V_ext, Wo):
    def body(x_ref, wq_ref, k_hbm, v_hbm, wo_ref, out_ref,
             xg, acc, kbuf, vbuf, qbuf, sendbuf, recvbuf,
             ag_send, ag_recv, rs_send, rs_recv, kv_sems):
        my = lax.axis_index("i")
        left = lax.rem(my - 1 + N_DEV, N_DEV)
        right = lax.rem(my + 1, N_DEV)
        h0 = my * HQ_PER

        def kv_dma(h, slot):
            ck = pltpu.make_async_copy(
                k_hbm.at[0, :, h0 + h, :], kbuf.at[slot], kv_sems.at[slot, 0])
            cv = pltpu.make_async_copy(
                v_hbm.at[0, :, h0 + h, :], vbuf.at[slot], kv_sems.at[slot, 1])
            return ck, cv

        k0, v0 = kv_dma(0, 0)
        k0.start()
        v0.start()

        xg[pl.ds(my, 1)] = x_ref[:].astype(BF16)

        barrier = pltpu.get_barrier_semaphore()
        for nbr in (left, right):
            pl.semaphore_signal(barrier, inc=1, device_id=(nbr,),
                                device_id_type=pl.DeviceIdType.MESH)
        pl.semaphore_wait(barrier, 2)

        wqb = wq_ref[:].astype(BF16)
        kb = lax.broadcasted_iota(jnp.int32, (SQ_PER, SKV), 1) // QBLK

        def rs_desc(t):
            return pltpu.make_async_remote_copy(
                src_ref=sendbuf.at[t % 2],
                dst_ref=recvbuf.at[t],
                send_sem=rs_send.at[t],
                recv_sem=rs_recv.at[t],
                device_id=(right,),
                device_id_type=pl.DeviceIdType.MESH,
            )

        for s in range(N_DEV):
            o = lax.rem(my - s + N_DEV, N_DEV)
            if s < N_DEV - 1:
                ag = pltpu.make_async_remote_copy(
                    src_ref=xg.at[pl.ds(o, 1)],
                    dst_ref=xg.at[pl.ds(o, 1)],
                    send_sem=ag_send.at[s],
                    recv_sem=ag_recv.at[s],
                    device_id=(right,),
                    device_id_type=pl.DeviceIdType.MESH,
                )
                ag.start()

            qbuf[:] = jnp.dot(xg[pl.ds(o, 1)][0], wqb,
                              preferred_element_type=F32).astype(BF16)
            qb = (o * (SQ_PER // QBLK)
                  + lax.broadcasted_iota(jnp.int32, (SQ_PER, SKV), 0) // QBLK)
            maskf = jnp.where(
                (qb == kb) | (kb == 0) | (lax.rem(qb + kb, 3) == 0),
                1.0, 0.0).astype(F32)
            acc[pl.ds(o, 1)] = jnp.zeros((1, SQ_PER, DM), F32)

            def hstep(h, carry, s=s, o=o, maskf=maskf):
                slot = lax.rem(h, 2)
                ck, cv = kv_dma(h, slot)
                ck.wait()
                cv.wait()
                if s < N_DEV - 1:
                    hn = lax.rem(h + 1, HQ_PER)
                    ckn, cvn = kv_dma(hn, 1 - slot)
                    ckn.start()
                    cvn.start()
                else:
                    @pl.when(h < HQ_PER - 1)
                    def _():
                        ckn, cvn = kv_dma(h + 1, 1 - slot)
                        ckn.start()
                        cvn.start()
                kh = kbuf[slot].astype(BF16)
                vh = vbuf[slot].astype(BF16)
                qh = qbuf[:, pl.ds(h * DH, DH)]
                sc = lax.dot_general(
                    qh, kh, (((1,), (1,)), ((), ())),
                    preferred_element_type=F32) * SCALE
                w = jnp.exp(sc) * maskf
                wsum = jnp.sum(w, axis=1, keepdims=True)
                ctx_h = jnp.dot(w.astype(BF16), vh,
                                preferred_element_type=F32) / wsum
                woh = wo_ref[pl.ds(h * DH, DH), :].astype(BF16)
                part = jnp.dot(ctx_h.astype(BF16), woh,
                               preferred_element_type=F32)
                acc[pl.ds(o, 1)] = acc[pl.ds(o, 1)] + part[None]
                return carry

            lax.fori_loop(0, HQ_PER, hstep, 0)

            if s >= 1:
                t = s - 1
                if t >= 1:
                    rs_desc(t - 1).wait_recv()
                if t >= 2:
                    rs_desc(t - 2).wait_send()
                data = acc[pl.ds(o, 1)]
                if t > 0:
                    data = data + recvbuf[pl.ds(t - 1, 1)]
                sendbuf[pl.ds(t % 2, 1)] = data
                rs_desc(t).start()

            if s < N_DEV - 1:
                ag.wait()

        rs_desc(N_DEV - 3).wait_send()
        last = rs_desc(N_DEV - 2)
        last.wait_send()
        last.wait_recv()
        out_ref[:] = acc[pl.ds(my, 1)] + recvbuf[pl.ds(N_DEV - 2, 1)]

    return pl.pallas_call(
        body,
        out_shape=jax.ShapeDtypeStruct((1, SQ_PER, DM), F32),
        in_specs=[
            pl.BlockSpec(memory_space=pltpu.VMEM),
            pl.BlockSpec(memory_space=pltpu.VMEM),
            pl.BlockSpec(memory_space=pl.ANY),
            pl.BlockSpec(memory_space=pl.ANY),
            pl.BlockSpec(memory_space=pltpu.VMEM),
        ],
        out_specs=pl.BlockSpec(memory_space=pltpu.VMEM),
        scratch_shapes=[
            pltpu.VMEM((N_DEV, SQ_PER, DM), BF16),
            pltpu.VMEM((N_DEV, SQ_PER, DM), F32),
            pltpu.VMEM((2, SKV, DH), F32),
            pltpu.VMEM((2, SKV, DH), F32),
            pltpu.VMEM((SQ_PER, DM), BF16),
            pltpu.VMEM((2, SQ_PER, DM), F32),
            pltpu.VMEM((N_DEV - 1, SQ_PER, DM), F32),
            pltpu.SemaphoreType.DMA((N_DEV - 1,)),
            pltpu.SemaphoreType.DMA((N_DEV - 1,)),
            pltpu.SemaphoreType.DMA((N_DEV - 1,)),
            pltpu.SemaphoreType.DMA((N_DEV - 1,)),
            pltpu.SemaphoreType.DMA((2, 2)),
        ],
        compiler_params=pltpu.CompilerParams(
            collective_id=0,
            vmem_limit_bytes=60 * 1024 * 1024,
        ),
    )(x, Wq, K_ext, V_ext, Wo)
